# Initial kernel scaffold; baseline (speedup 1.0000x reference)
import jax
import jax.numpy as jnp
from jax.experimental import pallas as pl


def kernel(x, attn0, attn1):
    raise NotImplementedError("write your pallas kernel here")


# speedup vs baseline: 1.3393x; 1.3393x over previous
import jax
import jax.numpy as jnp
from jax.experimental import pallas as pl
from jax.experimental.pallas import tpu as pltpu

_B = 2
_NW = 8
_NH = 16
_L = 256
_C = 512
_NSEL = 2 * _NH


def _fsm_body(x_ref, a0_ref, a1_ref, out_ref, acc_ref):
    b = pl.program_id(0)
    w = pl.program_id(1)

    @pl.when((b == 0) & (w == 0))
    def _init():
        acc_ref[...] = jnp.zeros_like(acc_ref)

    for a, ref in enumerate((a0_ref, a1_ref)):
        sums = [jnp.sum(ref[0, h], axis=0, keepdims=True) for h in range(_NH)]
        colsum = jnp.concatenate(sums, axis=0)
        base = a * (_B * _NH) + b * _NH
        acc_ref[pl.ds(base, _NH), :] += colsum

    @pl.when((b == _B - 1) & (w == _NW - 1))
    def _finish():
        acc = acc_ref[...]
        maxv = jnp.max(acc, axis=1, keepdims=True)
        iota = jax.lax.broadcasted_iota(jnp.int32, (2 * _B * _NH, _L), 1)
        idx = jnp.min(jnp.where(acc >= maxv, iota, _L), axis=1, keepdims=True)
        onehot = (iota == idx).astype(jnp.float32)
        for bb in range(_B):
            rows = (onehot[bb * _NH:(bb + 1) * _NH]
                    + onehot[_B * _NH + bb * _NH:_B * _NH + (bb + 1) * _NH])
            wgt = jnp.sum(rows, axis=0) * (1.0 / _NSEL)
            xb = x_ref[bb]
            out_ref[bb, :] = jnp.sum(xb * wgt.reshape(_L, 1), axis=0)


def kernel(x, attn0, attn1):
    grid = (_B, _NW)
    return pl.pallas_call(
        _fsm_body,
        grid=grid,
        in_specs=[
            pl.BlockSpec((_B, _L, _C), lambda b, w: (0, 0, 0)),
            pl.BlockSpec((1, _NH, _L, _L), lambda b, w: (b * _NW + w, 0, 0, 0)),
            pl.BlockSpec((1, _NH, _L, _L), lambda b, w: (b * _NW + w, 0, 0, 0)),
        ],
        out_specs=pl.BlockSpec((_B, _C), lambda b, w: (0, 0)),
        out_shape=jax.ShapeDtypeStruct((_B, _C), jnp.float32),
        scratch_shapes=[pltpu.VMEM((2 * _B * _NH, _L), jnp.float32)],
        compiler_params=pltpu.CompilerParams(
            dimension_semantics=("arbitrary", "arbitrary"),
        ),
    )(x, attn0, attn1)
